# Initial kernel scaffold; baseline (speedup 1.0000x reference)
import jax
import jax.numpy as jnp
from jax.experimental import pallas as pl


def kernel(pred, orig_img):
    raise NotImplementedError("write your pallas kernel here")


# speedup vs baseline: 1.9843x; 1.9843x over previous
import functools

import jax
import jax.numpy as jnp
from jax.experimental import pallas as pl
from jax.experimental.pallas import tpu as pltpu

_MAXD = 300
_LANES = 304
_R, _C = 160, 128
_CONF = 0.5
_IOU = 0.4


def _nms_body(x_ref, o_ref, cls_ref, *, wf, hf):
    s = x_ref[4]
    cls = jnp.zeros((_R, _C), jnp.float32)
    for c in range(1, 80):
        xc = x_ref[4 + c]
        upd = xc > s
        s = jnp.where(upd, xc, s)
        cls = jnp.where(upd, jnp.float32(c), cls)
    cls_ref[...] = cls

    flat = (jax.lax.broadcasted_iota(jnp.int32, (_R, _C), 0) * _C
            + jax.lax.broadcasted_iota(jnp.int32, (_R, _C), 1))
    lane = jax.lax.broadcasted_iota(jnp.int32, (1, _LANES), 1)
    lane128 = jax.lax.broadcasted_iota(jnp.int32, (1, _C), 1)
    zrow = jnp.zeros((1, _LANES), jnp.float32)

    def sel_body(k, carry):
        s, xr, yr, wr, hr, clsr, scr = carry
        m = jnp.max(s)
        fi = jnp.min(jnp.where(s == m, flat, jnp.int32(2 ** 30)))
        r = fi // _C
        c = fi % _C
        onek = (lane == k).astype(jnp.float32)
        cmask = (lane128 == c).astype(jnp.float32)

        def ext(row):
            return jnp.sum(row * cmask)

        vx = ext(x_ref[0, pl.ds(r, 1), :])
        vy = ext(x_ref[1, pl.ds(r, 1), :])
        vw = ext(x_ref[2, pl.ds(r, 1), :])
        vh = ext(x_ref[3, pl.ds(r, 1), :])
        vc = ext(cls_ref[pl.ds(r, 1), :])
        xr = xr + vx * onek
        yr = yr + vy * onek
        wr = wr + vw * onek
        hr = hr + vh * onek
        clsr = clsr + vc * onek
        scr = scr + m * onek
        s = jnp.where(flat == fi, -jnp.inf, s)
        return s, xr, yr, wr, hr, clsr, scr

    init = (s, zrow, zrow, zrow, zrow, zrow, zrow)
    _, xr, yr, wr, hr, clsr, scr = jax.lax.fori_loop(0, _MAXD, sel_body, init)

    x0r = xr - wr / 2
    y0r = yr - hr / 2
    x1r = xr + wr / 2
    y1r = yr + hr / 2
    ar = (x1r - x0r) * (y1r - y0r)
    valid = ((scr > _CONF) & (lane < _MAXD)).astype(jnp.float32)

    def nms_body(i, kr):
        sel = (lane == i).astype(jnp.float32)
        xi0 = jnp.sum(x0r * sel)
        yi0 = jnp.sum(y0r * sel)
        xi1 = jnp.sum(x1r * sel)
        yi1 = jnp.sum(y1r * sel)
        ai = jnp.sum(ar * sel)
        ki = jnp.sum(kr * sel)
        iw = jnp.maximum(jnp.minimum(x1r, xi1) - jnp.maximum(x0r, xi0), 0.0)
        ih = jnp.maximum(jnp.minimum(y1r, yi1) - jnp.maximum(y0r, yi0), 0.0)
        inter = iw * ih
        iou = inter / (ai + ar - inter + 1e-9)
        sup = ((iou > _IOU) & (lane > i)).astype(jnp.float32) * ki
        return kr * (1.0 - sup)

    kr = jax.lax.fori_loop(0, _MAXD, nms_body, valid)

    rows = jnp.concatenate(
        [
            jnp.clip(x0r, 0.0, wf) * kr,
            jnp.clip(y0r, 0.0, hf) * kr,
            jnp.clip(x1r, 0.0, wf) * kr,
            jnp.clip(y1r, 0.0, hf) * kr,
            scr * kr,
            clsr * kr,
            zrow,
            zrow,
        ],
        axis=0,
    )
    o_ref[...] = rows


def kernel(pred, orig_img):
    H = orig_img.shape[1]
    W = orig_img.shape[2]
    p = pred[0]
    n = p.shape[1]
    xp = jnp.pad(p, ((0, 0), (0, _R * _C - n)), constant_values=-jnp.inf)
    xp = xp.reshape(84, _R, _C)
    out = pl.pallas_call(
        functools.partial(_nms_body, wf=float(W), hf=float(H)),
        out_shape=jax.ShapeDtypeStruct((8, _LANES), jnp.float32),
        scratch_shapes=[pltpu.VMEM((_R, _C), jnp.float32)],
    )(xp)
    return out[:6, :_MAXD].T
